# baseline (speedup 1.0000x reference)
import functools

import jax
import jax.numpy as jnp
from jax import lax
from jax.experimental import pallas as pl
from jax.experimental.pallas import tpu as pltpu
from jax.experimental.pallas import tpu_sc as plsc

BATCH = 4096
MAXLEN = 200
EMBED = 64
LANES = 16

_info = plsc.get_sparse_core_info()
NC, NS = _info.num_cores, _info.num_subcores
NW = NC * NS
BPW = BATCH // NW
GRP = BPW // LANES


def _body(xt_hbm, pos_hbm, tok_hbm, out_hbm,
          idx_v, pos_v, rows0, rows1, obuf, g0, g1):
    w = lax.axis_index("s") * NC + lax.axis_index("c")
    b0 = w * BPW
    pltpu.sync_copy(xt_hbm.at[:, pl.ds(b0, BPW)], idx_v)
    pltpu.sync_copy(pos_hbm, pos_v)
    rows = (rows0, rows1)
    sems = (g0, g1)
    iota = lax.iota(jnp.int32, LANES)
    rowids = [iota + g * LANES for g in range(GRP)]

    def fetch_start(t, b):
        pltpu.make_async_copy(tok_hbm.at[idx_v.at[t]], rows[b], sems[b]).start()

    def fetch_wait(t, b):
        pltpu.make_async_copy(tok_hbm.at[idx_v.at[t]], rows[b], sems[b]).wait()

    def process(t, b):
        rbuf = rows[b]
        tv = jnp.full((LANES,), t, jnp.int32)

        def col(d, carry):
            dv = jnp.full((LANES,), d, jnp.int32)
            posd = plsc.load_gather(pos_v, [tv, dv])
            for g in range(GRP):
                v = plsc.load_gather(rbuf, [rowids[g], dv])
                obuf[d, pl.ds(g * LANES, LANES)] = v + posd
            return carry

        lax.fori_loop(0, EMBED, col, 0, unroll=2)

    def store(t):
        pltpu.sync_copy(obuf, out_hbm.at[t, :, pl.ds(b0, BPW)])

    fetch_start(0, 0)

    def outer(i, carry):
        t0 = i * 2
        fetch_start(t0 + 1, 1)
        fetch_wait(t0, 0)
        process(t0, 0)
        store(t0)

        @pl.when(t0 + 2 < MAXLEN)
        def _():
            fetch_start(t0 + 2, 0)

        fetch_wait(t0 + 1, 1)
        process(t0 + 1, 1)
        store(t0 + 1)
        return carry

    lax.fori_loop(0, MAXLEN // 2, outer, 0)


@jax.jit
def kernel(x, token_table, pos_table):
    B, T = x.shape
    V, D = token_table.shape
    assert (B, T, D) == (BATCH, MAXLEN, EMBED)
    xt = x.astype(jnp.int32).T

    run = pl.kernel(
        _body,
        out_type=jax.ShapeDtypeStruct((T, D, B), jnp.float32),
        mesh=plsc.VectorSubcoreMesh(core_axis_name="c", subcore_axis_name="s"),
        compiler_params=pltpu.CompilerParams(
            use_tc_tiling_on_sc=False, needs_layout_passes=False
        ),
        scratch_types=[
            pltpu.VMEM((MAXLEN, BPW), jnp.int32),
            pltpu.VMEM((MAXLEN, EMBED), jnp.float32),
            pltpu.VMEM((BPW, EMBED), jnp.float32),
            pltpu.VMEM((BPW, EMBED), jnp.float32),
            pltpu.VMEM((EMBED, BPW), jnp.float32),
            pltpu.SemaphoreType.DMA,
            pltpu.SemaphoreType.DMA,
        ],
    )
    out = run(xt, pos_table, token_table)
    return out.transpose(2, 0, 1)


# speedup vs baseline: 1.0928x; 1.0928x over previous
import functools

import jax
import jax.numpy as jnp
from jax import lax
from jax.experimental import pallas as pl
from jax.experimental.pallas import tpu as pltpu
from jax.experimental.pallas import tpu_sc as plsc

BATCH = 4096
MAXLEN = 200
EMBED = 64
LANES = 16

_info = plsc.get_sparse_core_info()
NC, NS = _info.num_cores, _info.num_subcores
NW = NC * NS
BPW = BATCH // NW
GRP = BPW // LANES


def _body(xt_hbm, pos_hbm, tok_hbm, out_hbm,
          idx_v, pos_v, rows0, rows1, obuf, g0, g1):
    w = lax.axis_index("s") * NC + lax.axis_index("c")
    b0 = w * BPW
    pltpu.sync_copy(xt_hbm.at[:, pl.ds(b0, BPW)], idx_v)
    pltpu.sync_copy(pos_hbm, pos_v)
    rows = (rows0, rows1)
    sems = (g0, g1)
    iota = lax.iota(jnp.int32, LANES)
    rowids = [iota + g * LANES for g in range(GRP)]

    def fetch_start(t, b):
        pltpu.make_async_copy(tok_hbm.at[idx_v.at[t]], rows[b], sems[b]).start()

    def fetch_wait(t, b):
        pltpu.make_async_copy(tok_hbm.at[idx_v.at[t]], rows[b], sems[b]).wait()

    def process(t, b):
        rbuf = rows[b]
        tv = jnp.full((LANES,), t, jnp.int32)

        def col(d, dv):
            posd = plsc.load_gather(pos_v, [tv, dv])
            dh = lax.shift_right_logical(d, 3)
            dl = lax.bitwise_and(d, 7)
            for g in range(GRP):
                v = plsc.load_gather(rbuf, [rowids[g], dv])
                obuf[dh, dl, pl.ds(g * LANES, LANES)] = v + posd
            return dv + 1

        lax.fori_loop(0, EMBED, col, jnp.zeros((LANES,), jnp.int32), unroll=4)

    def store(t):
        pltpu.sync_copy(obuf, out_hbm.at[t, :, w])

    fetch_start(0, 0)

    def outer(i, carry):
        t0 = i * 2
        fetch_start(t0 + 1, 1)
        fetch_wait(t0, 0)
        process(t0, 0)
        store(t0)

        @pl.when(t0 + 2 < MAXLEN)
        def _():
            fetch_start(t0 + 2, 0)

        fetch_wait(t0 + 1, 1)
        process(t0 + 1, 1)
        store(t0 + 1)
        return carry

    lax.fori_loop(0, MAXLEN // 2, outer, 0)


@jax.jit
def kernel(x, token_table, pos_table):
    B, T = x.shape
    V, D = token_table.shape
    assert (B, T, D) == (BATCH, MAXLEN, EMBED)
    xt = x.astype(jnp.int32).T

    run = pl.kernel(
        _body,
        out_type=jax.ShapeDtypeStruct((T, D // 8, NW, 8, BPW), jnp.float32),
        mesh=plsc.VectorSubcoreMesh(core_axis_name="c", subcore_axis_name="s"),
        compiler_params=pltpu.CompilerParams(
            use_tc_tiling_on_sc=False, needs_layout_passes=False
        ),
        scratch_types=[
            pltpu.VMEM((MAXLEN, BPW), jnp.int32),
            pltpu.VMEM((MAXLEN, EMBED), jnp.float32),
            pltpu.VMEM((BPW, EMBED), jnp.float32),
            pltpu.VMEM((BPW, EMBED), jnp.float32),
            pltpu.VMEM((EMBED // 8, 8, BPW), jnp.float32),
            pltpu.SemaphoreType.DMA,
            pltpu.SemaphoreType.DMA,
        ],
    )
    out = run(xt, pos_table, token_table)
    return out.transpose(2, 4, 0, 1, 3).reshape(B, T, D)


# speedup vs baseline: 1.7371x; 1.5896x over previous
import functools

import jax
import jax.numpy as jnp
from jax import lax
from jax.experimental import pallas as pl
from jax.experimental.pallas import tpu as pltpu
from jax.experimental.pallas import tpu_sc as plsc

BATCH = 4096
MAXLEN = 200
EMBED = 64
LANES = 16

_info = plsc.get_sparse_core_info()
NC, NS = _info.num_cores, _info.num_subcores
NW = NC * NS
BPW = BATCH // NW
VPR = EMBED // LANES


def _body(x_hbm, pos_hbm, tok_hbm, out_hbm, idx_v, pos_v, rows0, rows1, g0, g1):
    w = lax.axis_index("s") * NC + lax.axis_index("c")
    pltpu.sync_copy(x_hbm.at[w], idx_v)
    pltpu.sync_copy(pos_hbm, pos_v)
    rows = (rows0, rows1)
    sems = (g0, g1)

    def gather_start(t, b):
        pltpu.make_async_copy(tok_hbm.at[idx_v.at[t]], rows[b], sems[b]).start()

    def gather_wait(b):
        pltpu.make_async_copy(tok_hbm.at[idx_v.at[0]], rows[b], sems[b]).wait()

    def add_pos(t, b):
        rbuf = rows[b]
        pv = [pos_v[t, pl.ds(k * LANES, LANES)] for k in range(VPR)]

        def row(r, carry):
            for k in range(VPR):
                sl = pl.ds(k * LANES, LANES)
                rbuf[r, sl] = rbuf[r, sl] + pv[k]
            return carry

        lax.fori_loop(0, BPW, row, 0, unroll=4)

    def store(t, b):
        pltpu.sync_copy(rows[b], out_hbm.at[w, :, t])

    gather_start(0, 0)

    def outer(i, carry):
        t0 = i * 2
        gather_start(t0 + 1, 1)
        gather_wait(0)
        add_pos(t0, 0)
        store(t0, 0)

        @pl.when(t0 + 2 < MAXLEN)
        def _():
            gather_start(t0 + 2, 0)

        gather_wait(1)
        add_pos(t0 + 1, 1)
        store(t0 + 1, 1)
        return carry

    lax.fori_loop(0, MAXLEN // 2, outer, 0)


@jax.jit
def kernel(x, token_table, pos_table):
    B, T = x.shape
    V, D = token_table.shape
    assert (B, T, D) == (BATCH, MAXLEN, EMBED)
    x32 = x.astype(jnp.int32).reshape(NW, BPW, T).transpose(0, 2, 1)

    run = pl.kernel(
        _body,
        out_type=jax.ShapeDtypeStruct((NW, BPW, T, D), jnp.float32),
        mesh=plsc.VectorSubcoreMesh(core_axis_name="c", subcore_axis_name="s"),
        compiler_params=pltpu.CompilerParams(use_tc_tiling_on_sc=False),
        scratch_types=[
            pltpu.VMEM((T, BPW), jnp.int32),
            pltpu.VMEM((T, EMBED), jnp.float32),
            pltpu.VMEM((BPW, EMBED), jnp.float32),
            pltpu.VMEM((BPW, EMBED), jnp.float32),
            pltpu.SemaphoreType.DMA,
            pltpu.SemaphoreType.DMA,
        ],
    )
    out = run(x32, pos_table, token_table)
    return out.reshape(B, T, D)
